# Initial kernel scaffold; baseline (speedup 1.0000x reference)
import jax
import jax.numpy as jnp
from jax.experimental import pallas as pl


def kernel(node_ids, hyper_edge_index, entity_table, theta, conv_bias, rec_bias_b):
    raise NotImplementedError("write your pallas kernel here")


# speedup vs baseline: 16.5285x; 16.5285x over previous
import functools

import jax
import jax.numpy as jnp
from jax import lax
from jax.experimental import pallas as pl
from jax.experimental.pallas import tpu as pltpu
from jax.experimental.pallas import tpu_sc as plsc

N_ENTITY = 100000
EMB = 128
N_NODES = 10000
N_HYPEREDGES = 10000
E_TOTAL = 320000

NC = 2
NS = 16
NW = NC * NS

NB = 10240
DEAD = NB - 1
SLICE = NB // NS

K = 160
NCH = 79


def _sc_body(nodes_hbm, edges_hbm, ids_hbm, table_hbm, out_hbm,
             nidx, eidx, vals, tmp, tmp2, gidx, rows, cvec, accv,
             a_deg, a_cnt, a_s, a_c, sem):
    c = lax.axis_index("c")
    s = lax.axis_index("s")
    wid = c * NS + s

    one16 = jnp.full((16,), 1.0, jnp.float32)
    zero16 = jnp.zeros((16,), jnp.float32)

    pltpu.sync_copy(nodes_hbm.at[s], nidx)
    pltpu.sync_copy(edges_hbm.at[s], eidx)

    def fill_ones(j, _):
        for kk in range(8):
            vals[j, pl.ds(kk * 16, 16)] = one16
        return 0
    lax.fori_loop(0, K, fill_ones, 0)

    def fill_zero(k, _):
        tmp[pl.ds(k * 16, 16)] = zero16
        return 0
    lax.fori_loop(0, SLICE // 16, fill_zero, 0)

    sl = pl.ds(s * SLICE, SLICE)
    pltpu.sync_copy(tmp, a_deg.at[sl])
    pltpu.sync_copy(tmp, a_cnt.at[sl])
    pltpu.sync_copy(tmp, a_s.at[sl])
    pltpu.sync_copy(tmp, a_c.at[sl])
    plsc.subcore_barrier()

    def hist_group(g, _):
        cps = []
        for jj in range(4):
            j = g * 4 + jj
            cps.append(pltpu.async_copy(vals.at[j], a_deg.at[nidx.at[j]], sem, add=True))
            cps.append(pltpu.async_copy(vals.at[j], a_cnt.at[eidx.at[j]], sem, add=True))
        for cp in cps:
            cp.wait()
        return 0
    lax.fori_loop(0, K // 4, hist_group, 0)
    plsc.subcore_barrier()

    pltpu.sync_copy(a_deg.at[sl], tmp)

    def inv_row(k, _):
        d = tmp[pl.ds(k * 16, 16)]
        tmp[pl.ds(k * 16, 16)] = 1.0 / jnp.maximum(d, 1.0)
        return 0
    lax.fori_loop(0, SLICE // 16, inv_row, 0)
    pltpu.sync_copy(tmp, a_deg.at[sl])
    plsc.subcore_barrier()

    def gather_invdeg(g, _):
        cps = [pltpu.async_copy(a_deg.at[nidx.at[g * 8 + jj]], vals.at[g * 8 + jj], sem)
               for jj in range(8)]
        for cp in cps:
            cp.wait()
        return 0
    lax.fori_loop(0, K // 8, gather_invdeg, 0)

    def scatter_s(g, _):
        cps = [pltpu.async_copy(vals.at[g * 8 + jj], a_s.at[eidx.at[g * 8 + jj]], sem, add=True)
               for jj in range(8)]
        for cp in cps:
            cp.wait()
        return 0
    lax.fori_loop(0, K // 8, scatter_s, 0)
    plsc.subcore_barrier()

    pltpu.sync_copy(a_s.at[sl], tmp)
    pltpu.sync_copy(a_cnt.at[sl], tmp2)

    def w_row(k, _):
        sv = tmp[pl.ds(k * 16, 16)]
        cv = tmp2[pl.ds(k * 16, 16)]
        tmp[pl.ds(k * 16, 16)] = sv / jnp.maximum(cv, 1.0)
        return 0
    lax.fori_loop(0, SLICE // 16, w_row, 0)
    pltpu.sync_copy(tmp, a_s.at[sl])
    plsc.subcore_barrier()

    def gather_w(g, _):
        cps = [pltpu.async_copy(a_s.at[eidx.at[g * 8 + jj]], vals.at[g * 8 + jj], sem)
               for jj in range(8)]
        for cp in cps:
            cp.wait()
        return 0
    lax.fori_loop(0, K // 8, gather_w, 0)

    def scatter_c(g, _):
        cps = [pltpu.async_copy(vals.at[g * 8 + jj], a_c.at[nidx.at[g * 8 + jj]], sem, add=True)
               for jj in range(8)]
        for cp in cps:
            cp.wait()
        return 0
    lax.fori_loop(0, K // 8, scatter_c, 0)
    plsc.subcore_barrier()

    for kk in range(8):
        accv[pl.ds(kk * 16, 16)] = zero16

    def do_chunk(ch):
        pltpu.sync_copy(ids_hbm.at[ch], gidx)
        pltpu.async_copy(table_hbm.at[gidx], rows, sem).wait()
        pltpu.sync_copy(a_c.at[pl.ds(ch * 128, 128)], cvec)

        def grp_fma(m, accs):
            cw = cvec[pl.ds(m * 16, 16)]
            for r in range(16):
                cb = jnp.full((16,), cw[r], jnp.float32)
                i = m * 16 + r
                accs = tuple(accs[kk] + cb * rows[i, pl.ds(kk * 16, 16)]
                             for kk in range(8))
            return accs
        accs = lax.fori_loop(
            0, 8, grp_fma,
            tuple(accv[pl.ds(kk * 16, 16)] for kk in range(8)))
        for kk in range(8):
            accv[pl.ds(kk * 16, 16)] = accs[kk]

    do_chunk(wid)
    do_chunk(wid + NW)

    @pl.when(wid < NCH - 2 * NW)
    def _():
        do_chunk(wid + 2 * NW)

    pltpu.sync_copy(accv, out_hbm.at[wid])


@functools.cache
def _sc_compute():
  return functools.partial(
    pl.kernel,
    out_type=jax.ShapeDtypeStruct((NW, EMB), jnp.float32),
    mesh=plsc.VectorSubcoreMesh(core_axis_name="c", subcore_axis_name="s",
                                num_cores=NC, num_subcores=NS),
    scratch_types=[
        pltpu.VMEM((K, 128), jnp.int32),
        pltpu.VMEM((K, 128), jnp.int32),
        pltpu.VMEM((K, 128), jnp.float32),
        pltpu.VMEM((SLICE,), jnp.float32),
        pltpu.VMEM((SLICE,), jnp.float32),
        pltpu.VMEM((128,), jnp.int32),
        pltpu.VMEM((128, EMB), jnp.float32),
        pltpu.VMEM((128,), jnp.float32),
        pltpu.VMEM((EMB,), jnp.float32),
        pltpu.VMEM_SHARED((NB,), jnp.float32),
        pltpu.VMEM_SHARED((NB,), jnp.float32),
        pltpu.VMEM_SHARED((NB,), jnp.float32),
        pltpu.VMEM_SHARED((NB,), jnp.float32),
        pltpu.SemaphoreType.DMA,
    ],
  )(_sc_body)


ROWS_BLK = 2000
N_BLKS = N_ENTITY // ROWS_BLK


def _tc_body(pv_ref, th_ref, cb_ref, tb_ref, bb_ref, o_ref):
    v = jnp.sum(pv_ref[...], axis=0)
    u = jnp.dot(v, th_ref[...], preferred_element_type=jnp.float32)
    u = u * (1.0 / N_NODES) + cb_ref[...]
    s = jnp.dot(tb_ref[...], u, preferred_element_type=jnp.float32)
    o_ref[...] = s[None, None, :] + bb_ref[...]


_tc_scores = pl.pallas_call(
    _tc_body,
    grid=(N_BLKS,),
    in_specs=[
        pl.BlockSpec((NW, EMB), lambda i: (0, 0)),
        pl.BlockSpec((EMB, EMB), lambda i: (0, 0)),
        pl.BlockSpec((EMB,), lambda i: (0,)),
        pl.BlockSpec((ROWS_BLK, EMB), lambda i: (i, 0)),
        pl.BlockSpec((1, 1, ROWS_BLK), lambda i: (i, 0, 0)),
    ],
    out_specs=pl.BlockSpec((1, 1, ROWS_BLK), lambda i: (i, 0, 0)),
    out_shape=jax.ShapeDtypeStruct((N_BLKS, 1, ROWS_BLK), jnp.float32),
)


def kernel(node_ids, hyper_edge_index, entity_table, theta, conv_bias, rec_bias_b):
    nodes = hyper_edge_index[0].astype(jnp.int32)
    edges = hyper_edge_index[1].astype(jnp.int32)
    pad = NS * K * 128 - E_TOTAL
    nodes3 = jnp.concatenate(
        [nodes, jnp.full((pad,), DEAD, jnp.int32)]).reshape(NS, K, 128)
    edges3 = jnp.concatenate(
        [edges, jnp.full((pad,), DEAD, jnp.int32)]).reshape(NS, K, 128)
    ids2 = jnp.concatenate(
        [node_ids.astype(jnp.int32),
         jnp.zeros((NCH * 128 - N_NODES,), jnp.int32)]).reshape(NCH, 128)
    partials = _sc_compute()(nodes3, edges3, ids2, entity_table)
    bb2 = rec_bias_b.reshape(N_BLKS, 1, ROWS_BLK)
    scores2 = _tc_scores(partials, theta, conv_bias, entity_table, bb2)
    return scores2.reshape(N_ENTITY)


# speedup vs baseline: 16.5641x; 1.0022x over previous
import functools

import jax
import jax.numpy as jnp
from jax import lax
from jax.experimental import pallas as pl
from jax.experimental.pallas import tpu as pltpu
from jax.experimental.pallas import tpu_sc as plsc

N_ENTITY = 100000
EMB = 128
N_NODES = 10000
N_HYPEREDGES = 10000
E_TOTAL = 320000

NC = 2
NS = 16
NW = NC * NS

NB = 10240
DEAD = NB - 1
SLICE = NB // NS

EPT = 20480
NPW = 320


def _sc_body(nodes_hbm, edges_hbm, ids_hbm, table_hbm, out_hbm,
             nidx, eidx, vals, tmp, tmp2, gidx, rows, cvec, accv,
             a_deg, a_cnt, a_s, a_c, sem, gsem):
    c = lax.axis_index("c")
    s = lax.axis_index("s")
    wid = c * NS + s

    one16 = jnp.full((16,), 1.0, jnp.float32)
    zero16 = jnp.zeros((16,), jnp.float32)

    pltpu.sync_copy(ids_hbm.at[wid], gidx)
    rows_cp = pltpu.async_copy(table_hbm.at[gidx], rows, gsem)

    cp_n = pltpu.async_copy(nodes_hbm.at[s], nidx, sem)
    cp_e = pltpu.async_copy(edges_hbm.at[s], eidx, sem)

    def fill_ones(j, _):
        vals[pl.ds(j * 16, 16)] = one16
        return 0
    lax.fori_loop(0, EPT // 16, fill_ones, 0)

    def fill_zero(k, _):
        tmp[pl.ds(k * 16, 16)] = zero16
        return 0
    lax.fori_loop(0, SLICE // 16, fill_zero, 0)

    sl = pl.ds(s * SLICE, SLICE)
    pltpu.sync_copy(tmp, a_deg.at[sl])
    pltpu.sync_copy(tmp, a_cnt.at[sl])
    pltpu.sync_copy(tmp, a_s.at[sl])
    pltpu.sync_copy(tmp, a_c.at[sl])
    cp_n.wait()
    cp_e.wait()
    plsc.subcore_barrier()

    cp_d = pltpu.async_copy(vals, a_deg.at[nidx], sem, add=True)
    cp_c = pltpu.async_copy(vals, a_cnt.at[eidx], sem, add=True)
    cp_d.wait()
    cp_c.wait()
    plsc.subcore_barrier()

    pltpu.sync_copy(a_deg.at[sl], tmp)

    def inv_row(k, _):
        d = tmp[pl.ds(k * 16, 16)]
        tmp[pl.ds(k * 16, 16)] = 1.0 / jnp.maximum(d, 1.0)
        return 0
    lax.fori_loop(0, SLICE // 16, inv_row, 0)
    pltpu.sync_copy(tmp, a_deg.at[sl])
    plsc.subcore_barrier()

    pltpu.sync_copy(a_deg.at[nidx], vals)
    pltpu.sync_copy(vals, a_s.at[eidx], add=True)
    plsc.subcore_barrier()

    pltpu.sync_copy(a_s.at[sl], tmp)
    pltpu.sync_copy(a_cnt.at[sl], tmp2)

    def w_row(k, _):
        sv = tmp[pl.ds(k * 16, 16)]
        cv = tmp2[pl.ds(k * 16, 16)]
        tmp[pl.ds(k * 16, 16)] = sv / jnp.maximum(cv, 1.0)
        return 0
    lax.fori_loop(0, SLICE // 16, w_row, 0)
    pltpu.sync_copy(tmp, a_s.at[sl])
    plsc.subcore_barrier()

    pltpu.sync_copy(a_s.at[eidx], vals)
    pltpu.sync_copy(vals, a_c.at[nidx], add=True)
    plsc.subcore_barrier()

    @pl.when(s == NS - 1)
    def _():
        cvec[pl.ds(0, 16)] = zero16
        pltpu.sync_copy(cvec.at[pl.ds(0, 16)], a_c.at[pl.ds(NB - 16, 16)])
    plsc.subcore_barrier()

    pltpu.sync_copy(a_c.at[pl.ds(wid * NPW, NPW)], cvec)
    rows_cp.wait()

    def grp_fma(m, accs):
        cw = cvec[pl.ds(m * 16, 16)]
        for r in range(16):
            cb = jnp.full((16,), cw[r], jnp.float32)
            i = m * 16 + r
            accs = tuple(accs[kk] + cb * rows[i, pl.ds(kk * 16, 16)]
                         for kk in range(8))
        return accs
    accs = lax.fori_loop(0, NPW // 16, grp_fma,
                         tuple(zero16 for _ in range(8)))
    for kk in range(8):
        accv[pl.ds(kk * 16, 16)] = accs[kk]

    pltpu.sync_copy(accv, out_hbm.at[wid])


@functools.cache
def _sc_compute():
  return functools.partial(
    pl.kernel,
    out_type=jax.ShapeDtypeStruct((NW, EMB), jnp.float32),
    mesh=plsc.VectorSubcoreMesh(core_axis_name="c", subcore_axis_name="s",
                                num_cores=NC, num_subcores=NS),
    scratch_types=[
        pltpu.VMEM((EPT,), jnp.int32),
        pltpu.VMEM((EPT,), jnp.int32),
        pltpu.VMEM((EPT,), jnp.float32),
        pltpu.VMEM((SLICE,), jnp.float32),
        pltpu.VMEM((SLICE,), jnp.float32),
        pltpu.VMEM((NPW,), jnp.int32),
        pltpu.VMEM((NPW, EMB), jnp.float32),
        pltpu.VMEM((NPW,), jnp.float32),
        pltpu.VMEM((EMB,), jnp.float32),
        pltpu.VMEM_SHARED((NB,), jnp.float32),
        pltpu.VMEM_SHARED((NB,), jnp.float32),
        pltpu.VMEM_SHARED((NB,), jnp.float32),
        pltpu.VMEM_SHARED((NB,), jnp.float32),
        pltpu.SemaphoreType.DMA,
        pltpu.SemaphoreType.DMA,
    ],
  )(_sc_body)


ROWS_BLK = 2000
N_BLKS = N_ENTITY // ROWS_BLK


def _tc_body(pv_ref, th_ref, cb_ref, tb_ref, bb_ref, o_ref):
    v = jnp.sum(pv_ref[...], axis=0)
    u = jnp.dot(v, th_ref[...], preferred_element_type=jnp.float32)
    u = u * (1.0 / N_NODES) + cb_ref[...]
    s = jnp.dot(tb_ref[...], u, preferred_element_type=jnp.float32)
    o_ref[...] = s[None, None, :] + bb_ref[...]


_tc_scores = pl.pallas_call(
    _tc_body,
    grid=(N_BLKS,),
    in_specs=[
        pl.BlockSpec((NW, EMB), lambda i: (0, 0)),
        pl.BlockSpec((EMB, EMB), lambda i: (0, 0)),
        pl.BlockSpec((EMB,), lambda i: (0,)),
        pl.BlockSpec((ROWS_BLK, EMB), lambda i: (i, 0)),
        pl.BlockSpec((1, 1, ROWS_BLK), lambda i: (i, 0, 0)),
    ],
    out_specs=pl.BlockSpec((1, 1, ROWS_BLK), lambda i: (i, 0, 0)),
    out_shape=jax.ShapeDtypeStruct((N_BLKS, 1, ROWS_BLK), jnp.float32),
)


def kernel(node_ids, hyper_edge_index, entity_table, theta, conv_bias, rec_bias_b):
    nodes = hyper_edge_index[0].astype(jnp.int32)
    edges = hyper_edge_index[1].astype(jnp.int32)
    pad = NS * EPT - E_TOTAL
    nodes2 = jnp.concatenate(
        [nodes, jnp.full((pad,), DEAD, jnp.int32)]).reshape(NS, EPT)
    edges2 = jnp.concatenate(
        [edges, jnp.full((pad,), DEAD, jnp.int32)]).reshape(NS, EPT)
    ids2 = jnp.concatenate(
        [node_ids.astype(jnp.int32),
         jnp.zeros((NW * NPW - N_NODES,), jnp.int32)]).reshape(NW, NPW)
    partials = _sc_compute()(nodes2, edges2, ids2, entity_table)
    bb2 = rec_bias_b.reshape(N_BLKS, 1, ROWS_BLK)
    scores2 = _tc_scores(partials, theta, conv_bias, entity_table, bb2)
    return scores2.reshape(N_ENTITY)


# speedup vs baseline: 16.5886x; 1.0015x over previous
import functools

import jax
import jax.numpy as jnp
from jax import lax
from jax.experimental import pallas as pl
from jax.experimental.pallas import tpu as pltpu
from jax.experimental.pallas import tpu_sc as plsc

N_ENTITY = 100000
EMB = 128
N_NODES = 10000
N_HYPEREDGES = 10000
E_TOTAL = 320000

NC = 2
NS = 16
NW = NC * NS

NB = 10240
DEAD = NB - 1
SLICE = NB // NS

EPT = 20480
NPW = 320


def _sc_body(nodes_hbm, edges_hbm, ids_hbm, table_hbm, out_hbm,
             nidx, eidx, vals, tmp, tmp2, gidx, rows, cvec, accv,
             a_deg, a_cnt, a_s, a_c, sem, gsem):
    c = lax.axis_index("c")
    s = lax.axis_index("s")
    wid = c * NS + s

    one16 = jnp.full((16,), 1.0, jnp.float32)
    zero16 = jnp.zeros((16,), jnp.float32)

    pltpu.sync_copy(ids_hbm.at[wid], gidx)
    rows_cp = pltpu.async_copy(table_hbm.at[gidx], rows, gsem)

    with jax.named_scope("ph0_stage"):
        cp_n = pltpu.async_copy(nodes_hbm.at[s], nidx, sem)
        cp_e = pltpu.async_copy(edges_hbm.at[s], eidx, sem)

    with jax.named_scope("ph0_fill"):
        def fill_ones(j, _):
            vals[pl.ds(j * 16, 16)] = one16
            return 0
        lax.fori_loop(0, EPT // 16, fill_ones, 0)

        def fill_zero(k, _):
            tmp[pl.ds(k * 16, 16)] = zero16
            return 0
        lax.fori_loop(0, SLICE // 16, fill_zero, 0)

        sl = pl.ds(s * SLICE, SLICE)
        pltpu.sync_copy(tmp, a_deg.at[sl])
        pltpu.sync_copy(tmp, a_cnt.at[sl])
        pltpu.sync_copy(tmp, a_s.at[sl])
        pltpu.sync_copy(tmp, a_c.at[sl])
        cp_n.wait()
        cp_e.wait()
        plsc.subcore_barrier()

    with jax.named_scope("ph1_hist"):
        cp_d = pltpu.async_copy(vals, a_deg.at[nidx], sem, add=True)
        cp_c = pltpu.async_copy(vals, a_cnt.at[eidx], sem, add=True)
        cp_d.wait()
        cp_c.wait()
        plsc.subcore_barrier()

    with jax.named_scope("ph2_inv"):
        pltpu.sync_copy(a_deg.at[sl], tmp)

        def inv_row(k, _):
            d = tmp[pl.ds(k * 16, 16)]
            tmp[pl.ds(k * 16, 16)] = 1.0 / jnp.maximum(d, 1.0)
            return 0
        lax.fori_loop(0, SLICE // 16, inv_row, 0)
        pltpu.sync_copy(tmp, a_deg.at[sl])
        plsc.subcore_barrier()

    with jax.named_scope("ph3_gsc"):
        pltpu.sync_copy(a_deg.at[nidx], vals)
        pltpu.sync_copy(vals, a_s.at[eidx], add=True)
        plsc.subcore_barrier()

    with jax.named_scope("ph4_w"):
        pltpu.sync_copy(a_s.at[sl], tmp)
        pltpu.sync_copy(a_cnt.at[sl], tmp2)

        def w_row(k, _):
            sv = tmp[pl.ds(k * 16, 16)]
            cv = tmp2[pl.ds(k * 16, 16)]
            tmp[pl.ds(k * 16, 16)] = sv / jnp.maximum(cv, 1.0)
            return 0
        lax.fori_loop(0, SLICE // 16, w_row, 0)
        pltpu.sync_copy(tmp, a_s.at[sl])
        plsc.subcore_barrier()

    with jax.named_scope("ph5_gsc"):
        pltpu.sync_copy(a_s.at[eidx], vals)
        pltpu.sync_copy(vals, a_c.at[nidx], add=True)
        plsc.subcore_barrier()

    @pl.when(s == NS - 1)
    def _():
        cvec[pl.ds(0, 16)] = zero16
        pltpu.sync_copy(cvec.at[pl.ds(0, 16)], a_c.at[pl.ds(NB - 16, 16)])
    plsc.subcore_barrier()

    with jax.named_scope("ph6_cvec"):
        pltpu.sync_copy(a_c.at[pl.ds(wid * NPW, NPW)], cvec)
        rows_cp.wait()

    def grp_fma(m, accs):
        cw = cvec[pl.ds(m * 16, 16)]
        for r in range(16):
            cb = jnp.full((16,), cw[r], jnp.float32)
            i = m * 16 + r
            accs = tuple(accs[kk] + cb * rows[i, pl.ds(kk * 16, 16)]
                         for kk in range(8))
        return accs
    with jax.named_scope("ph6_fma"):
        accs = lax.fori_loop(0, NPW // 16, grp_fma,
                             tuple(zero16 for _ in range(8)))
        for kk in range(8):
            accv[pl.ds(kk * 16, 16)] = accs[kk]

        pltpu.sync_copy(accv, out_hbm.at[wid])


@functools.cache
def _sc_compute():
  return functools.partial(
    pl.kernel,
    out_type=jax.ShapeDtypeStruct((NW, EMB), jnp.float32),
    mesh=plsc.VectorSubcoreMesh(core_axis_name="c", subcore_axis_name="s",
                                num_cores=NC, num_subcores=NS),
    scratch_types=[
        pltpu.VMEM((EPT,), jnp.int32),
        pltpu.VMEM((EPT,), jnp.int32),
        pltpu.VMEM((EPT,), jnp.float32),
        pltpu.VMEM((SLICE,), jnp.float32),
        pltpu.VMEM((SLICE,), jnp.float32),
        pltpu.VMEM((NPW,), jnp.int32),
        pltpu.VMEM((NPW, EMB), jnp.float32),
        pltpu.VMEM((NPW,), jnp.float32),
        pltpu.VMEM((EMB,), jnp.float32),
        pltpu.VMEM_SHARED((NB,), jnp.float32),
        pltpu.VMEM_SHARED((NB,), jnp.float32),
        pltpu.VMEM_SHARED((NB,), jnp.float32),
        pltpu.VMEM_SHARED((NB,), jnp.float32),
        pltpu.SemaphoreType.DMA,
        pltpu.SemaphoreType.DMA,
    ],
  )(_sc_body)


ROWS_BLK = 2000
N_BLKS = N_ENTITY // ROWS_BLK


def _tc_body(pv_ref, th_ref, cb_ref, tb_ref, bb_ref, o_ref):
    v = jnp.sum(pv_ref[...], axis=0)
    u = jnp.dot(v, th_ref[...], preferred_element_type=jnp.float32)
    u = u * (1.0 / N_NODES) + cb_ref[...]
    s = jnp.dot(tb_ref[...], u, preferred_element_type=jnp.float32)
    o_ref[...] = s[None, None, :] + bb_ref[...]


_tc_scores = pl.pallas_call(
    _tc_body,
    grid=(N_BLKS,),
    in_specs=[
        pl.BlockSpec((NW, EMB), lambda i: (0, 0)),
        pl.BlockSpec((EMB, EMB), lambda i: (0, 0)),
        pl.BlockSpec((EMB,), lambda i: (0,)),
        pl.BlockSpec((ROWS_BLK, EMB), lambda i: (i, 0)),
        pl.BlockSpec((1, 1, ROWS_BLK), lambda i: (i, 0, 0)),
    ],
    out_specs=pl.BlockSpec((1, 1, ROWS_BLK), lambda i: (i, 0, 0)),
    out_shape=jax.ShapeDtypeStruct((N_BLKS, 1, ROWS_BLK), jnp.float32),
)


def kernel(node_ids, hyper_edge_index, entity_table, theta, conv_bias, rec_bias_b):
    nodes = hyper_edge_index[0].astype(jnp.int32)
    edges = hyper_edge_index[1].astype(jnp.int32)
    pad = NS * EPT - E_TOTAL
    nodes2 = jnp.concatenate(
        [nodes, jnp.full((pad,), DEAD, jnp.int32)]).reshape(NS, EPT)
    edges2 = jnp.concatenate(
        [edges, jnp.full((pad,), DEAD, jnp.int32)]).reshape(NS, EPT)
    ids2 = jnp.concatenate(
        [node_ids.astype(jnp.int32),
         jnp.zeros((NW * NPW - N_NODES,), jnp.int32)]).reshape(NW, NPW)
    partials = _sc_compute()(nodes2, edges2, ids2, entity_table)
    bb2 = rec_bias_b.reshape(N_BLKS, 1, ROWS_BLK)
    scores2 = _tc_scores(partials, theta, conv_bias, entity_table, bb2)
    return scores2.reshape(N_ENTITY)


# speedup vs baseline: 39.8877x; 2.4045x over previous
import functools

import jax
import jax.numpy as jnp
from jax import lax
from jax.experimental import pallas as pl
from jax.experimental.pallas import tpu as pltpu
from jax.experimental.pallas import tpu_sc as plsc

N_ENTITY = 100000
EMB = 128
N_NODES = 10000
N_HYPEREDGES = 10000
E_TOTAL = 320000

NC = 2
NS = 16
NW = NC * NS

NB = 10240
DEAD = NB - 1
SLICE = NB // NS

EPT = 20000
NPW = 320


def _sc_body(hei_hbm, ids_hbm, table_hbm, out_hbm,
             nidx, eidx, vals, tmp, tmp2, gidx, rows, cvec, accv,
             a_deg, a_cnt, a_s, a_c, sem, gsem):
    c = lax.axis_index("c")
    s = lax.axis_index("s")
    wid = c * NS + s

    one16 = jnp.full((16,), 1.0, jnp.float32)
    zero16 = jnp.zeros((16,), jnp.float32)

    with jax.named_scope("ph0_stage"):
        cp_n = pltpu.async_copy(hei_hbm.at[pl.ds(s * EPT, EPT)], nidx, sem)
        cp_e = pltpu.async_copy(hei_hbm.at[pl.ds(E_TOTAL + s * EPT, EPT)], eidx, sem)

    with jax.named_scope("ph0_fill"):
        def fill_ones(j, _):
            vals[pl.ds(j * 16, 16)] = one16
            return 0
        lax.fori_loop(0, EPT // 16, fill_ones, 0, unroll=10)

        def fill_zero(k, _):
            tmp[pl.ds(k * 16, 16)] = zero16
            return 0
        lax.fori_loop(0, SLICE // 16, fill_zero, 0, unroll=8)

        sl = pl.ds(s * SLICE, SLICE)
        pltpu.sync_copy(tmp, a_deg.at[sl])
        pltpu.sync_copy(tmp, a_cnt.at[sl])
        pltpu.sync_copy(tmp, a_s.at[sl])
        pltpu.sync_copy(tmp, a_c.at[sl])
        cp_n.wait()
        cp_e.wait()
        plsc.subcore_barrier()

    with jax.named_scope("ph1_hist"):
        cp_d = pltpu.async_copy(vals, a_deg.at[nidx], sem, add=True)
        cp_c = pltpu.async_copy(vals, a_cnt.at[eidx], sem, add=True)
        cp_d.wait()
        cp_c.wait()
        plsc.subcore_barrier()

    @pl.when(wid < NW - 1)
    def _():
        pltpu.sync_copy(ids_hbm.at[pl.ds(wid * NPW, NPW)], gidx)

    @pl.when(wid == NW - 1)
    def _():
        zero16i = jnp.zeros((16,), jnp.int32)

        def fill_pad(j, _):
            gidx[pl.ds(j * 16, 16)] = zero16i
            return 0
        lax.fori_loop(N_NODES // 16 - (NW - 1) * (NPW // 16), NPW // 16,
                      fill_pad, 0)
        pltpu.sync_copy(ids_hbm.at[pl.ds((NW - 1) * NPW, N_NODES - (NW - 1) * NPW)],
                        gidx.at[pl.ds(0, N_NODES - (NW - 1) * NPW)])
    rows_cp = pltpu.async_copy(table_hbm.at[gidx], rows, gsem)

    with jax.named_scope("ph2_inv"):
        pltpu.sync_copy(a_deg.at[sl], tmp)

        def inv_row(k, _):
            d = tmp[pl.ds(k * 16, 16)]
            tmp[pl.ds(k * 16, 16)] = 1.0 / jnp.maximum(d, 1.0)
            return 0
        lax.fori_loop(0, SLICE // 16, inv_row, 0)
        pltpu.sync_copy(tmp, a_deg.at[sl])
        plsc.subcore_barrier()

    with jax.named_scope("ph3_gsc"):
        pltpu.sync_copy(a_deg.at[nidx], vals)
        pltpu.sync_copy(vals, a_s.at[eidx], add=True)
        plsc.subcore_barrier()

    with jax.named_scope("ph4_w"):
        pltpu.sync_copy(a_s.at[sl], tmp)
        pltpu.sync_copy(a_cnt.at[sl], tmp2)

        def w_row(k, _):
            sv = tmp[pl.ds(k * 16, 16)]
            cv = tmp2[pl.ds(k * 16, 16)]
            tmp[pl.ds(k * 16, 16)] = sv / jnp.maximum(cv, 1.0)
            return 0
        lax.fori_loop(0, SLICE // 16, w_row, 0)
        pltpu.sync_copy(tmp, a_s.at[sl])
        plsc.subcore_barrier()

    with jax.named_scope("ph5_gsc"):
        pltpu.sync_copy(a_s.at[eidx], vals)
        pltpu.sync_copy(vals, a_c.at[nidx], add=True)
        plsc.subcore_barrier()

    with jax.named_scope("ph6_cvec"):
        pltpu.sync_copy(a_c.at[pl.ds(wid * NPW, NPW)], cvec)
        rows_cp.wait()

    def grp_fma(m, accs):
        cw = cvec[pl.ds(m * 16, 16)]
        for r in range(16):
            cb = jnp.full((16,), cw[r], jnp.float32)
            i = m * 16 + r
            accs = tuple(accs[kk] + cb * rows[i, pl.ds(kk * 16, 16)]
                         for kk in range(8))
        return accs
    with jax.named_scope("ph6_fma"):
        accs = lax.fori_loop(0, NPW // 16, grp_fma,
                             tuple(zero16 for _ in range(8)))
        for kk in range(8):
            accv[pl.ds(kk * 16, 16)] = accs[kk]

        pltpu.sync_copy(accv, out_hbm.at[wid])


@functools.cache
def _sc_compute():
  return functools.partial(
    pl.kernel,
    out_type=jax.ShapeDtypeStruct((NW, EMB), jnp.float32),
    mesh=plsc.VectorSubcoreMesh(core_axis_name="c", subcore_axis_name="s",
                                num_cores=NC, num_subcores=NS),
    scratch_types=[
        pltpu.VMEM((EPT,), jnp.int32),
        pltpu.VMEM((EPT,), jnp.int32),
        pltpu.VMEM((EPT,), jnp.float32),
        pltpu.VMEM((SLICE,), jnp.float32),
        pltpu.VMEM((SLICE,), jnp.float32),
        pltpu.VMEM((NPW,), jnp.int32),
        pltpu.VMEM((NPW, EMB), jnp.float32),
        pltpu.VMEM((NPW,), jnp.float32),
        pltpu.VMEM((EMB,), jnp.float32),
        pltpu.VMEM_SHARED((NB,), jnp.float32),
        pltpu.VMEM_SHARED((NB,), jnp.float32),
        pltpu.VMEM_SHARED((NB,), jnp.float32),
        pltpu.VMEM_SHARED((NB,), jnp.float32),
        pltpu.SemaphoreType.DMA,
        pltpu.SemaphoreType.DMA,
    ],
  )(_sc_body)


ROWS_BLK = 10000
N_BLKS = N_ENTITY // ROWS_BLK


def _tc_body(pv_ref, th_ref, cb_ref, tb_ref, bb_ref, o_ref, u_ref):
    @pl.when(pl.program_id(0) == 0)
    def _():
        v = jnp.sum(pv_ref[...], axis=0, keepdims=True)
        u = jnp.dot(v, th_ref[...], preferred_element_type=jnp.float32)
        u_ref[...] = u * (1.0 / N_NODES) + cb_ref[...][None, :]
    s2 = jax.lax.dot_general(u_ref[...], tb_ref[...],
                             (((1,), (1,)), ((), ())),
                             preferred_element_type=jnp.float32)
    o_ref[...] = s2[None] + bb_ref[...]


_tc_scores = pl.pallas_call(
    _tc_body,
    grid=(N_BLKS,),
    in_specs=[
        pl.BlockSpec((NW, EMB), lambda i: (0, 0)),
        pl.BlockSpec((EMB, EMB), lambda i: (0, 0)),
        pl.BlockSpec((EMB,), lambda i: (0,)),
        pl.BlockSpec((ROWS_BLK, EMB), lambda i: (i, 0)),
        pl.BlockSpec((1, 1, ROWS_BLK), lambda i: (i, 0, 0)),
    ],
    out_specs=pl.BlockSpec((1, 1, ROWS_BLK), lambda i: (i, 0, 0)),
    out_shape=jax.ShapeDtypeStruct((N_BLKS, 1, ROWS_BLK), jnp.float32),
    scratch_shapes=[pltpu.VMEM((1, EMB), jnp.float32)],
)


def kernel(node_ids, hyper_edge_index, entity_table, theta, conv_bias, rec_bias_b):
    hei_flat = hyper_edge_index.astype(jnp.int32).reshape(2 * E_TOTAL)
    partials = _sc_compute()(hei_flat, node_ids.astype(jnp.int32),
                             entity_table)
    bb2 = rec_bias_b.reshape(N_BLKS, 1, ROWS_BLK)
    scores2 = _tc_scores(partials, theta, conv_bias, entity_table, bb2)
    return scores2.reshape(N_ENTITY)


# speedup vs baseline: 40.3774x; 1.0123x over previous
import functools

import jax
import jax.numpy as jnp
from jax import lax
from jax.experimental import pallas as pl
from jax.experimental.pallas import tpu as pltpu
from jax.experimental.pallas import tpu_sc as plsc

N_ENTITY = 100000
EMB = 128
N_NODES = 10000
N_HYPEREDGES = 10000
E_TOTAL = 320000

NC = 2
NS = 16
NW = NC * NS

NB = 10240
DEAD = NB - 1
SLICE = NB // NS

EPT = 20000
NPW = 320


def _sc_body(hei_hbm, ids_hbm, table_hbm, out_hbm,
             nidx, eidx, vals, tmp, tmp2, gidx, rows, cvec, accv,
             a_deg, a_cnt, a_s, a_c, sem, gsem):
    c = lax.axis_index("c")
    s = lax.axis_index("s")
    wid = c * NS + s

    one16 = jnp.full((16,), 1.0, jnp.float32)
    zero16 = jnp.zeros((16,), jnp.float32)

    with jax.named_scope("ph0_stage"):
        cp_n = pltpu.async_copy(hei_hbm.at[pl.ds(s * EPT, EPT)], nidx, sem)
        cp_e = pltpu.async_copy(hei_hbm.at[pl.ds(E_TOTAL + s * EPT, EPT)], eidx, sem)

    with jax.named_scope("ph0_fill"):
        def fill_ones(j, _):
            vals[pl.ds(j * 16, 16)] = one16
            return 0
        lax.fori_loop(0, EPT // 16, fill_ones, 0, unroll=10)

        def fill_zero(k, _):
            tmp[pl.ds(k * 16, 16)] = zero16
            return 0
        lax.fori_loop(0, SLICE // 16, fill_zero, 0, unroll=8)

        sl = pl.ds(s * SLICE, SLICE)
        pltpu.sync_copy(tmp, a_deg.at[sl])
        pltpu.sync_copy(tmp, a_cnt.at[sl])
        pltpu.sync_copy(tmp, a_s.at[sl])
        pltpu.sync_copy(tmp, a_c.at[sl])
        cp_n.wait()
        cp_e.wait()
        plsc.subcore_barrier()

    with jax.named_scope("ph1_hist"):
        cp_d = pltpu.async_copy(vals, a_deg.at[nidx], sem, add=True)
        cp_c = pltpu.async_copy(vals, a_cnt.at[eidx], sem, add=True)
        cp_d.wait()
        plsc.subcore_barrier()

    @pl.when(wid < NW - 1)
    def _():
        pltpu.sync_copy(ids_hbm.at[pl.ds(wid * NPW, NPW)], gidx)

    @pl.when(wid == NW - 1)
    def _():
        zero16i = jnp.zeros((16,), jnp.int32)

        def fill_pad(j, _):
            gidx[pl.ds(j * 16, 16)] = zero16i
            return 0
        lax.fori_loop(N_NODES // 16 - (NW - 1) * (NPW // 16), NPW // 16,
                      fill_pad, 0)
        pltpu.sync_copy(ids_hbm.at[pl.ds((NW - 1) * NPW, N_NODES - (NW - 1) * NPW)],
                        gidx.at[pl.ds(0, N_NODES - (NW - 1) * NPW)])
    rows_cp = pltpu.async_copy(table_hbm.at[gidx], rows, gsem)

    with jax.named_scope("ph2_inv"):
        pltpu.sync_copy(a_deg.at[sl], tmp)

        def inv_row(k, _):
            d = tmp[pl.ds(k * 16, 16)]
            tmp[pl.ds(k * 16, 16)] = 1.0 / jnp.maximum(d, 1.0)
            return 0
        lax.fori_loop(0, SLICE // 16, inv_row, 0)
        pltpu.sync_copy(tmp, a_deg.at[sl])
        plsc.subcore_barrier()

    with jax.named_scope("ph3_gsc"):
        cp_c.wait()
        pltpu.sync_copy(a_deg.at[nidx], vals)
        pltpu.sync_copy(vals, a_s.at[eidx], add=True)
        plsc.subcore_barrier()

    with jax.named_scope("ph4_w"):
        pltpu.sync_copy(a_s.at[sl], tmp)
        pltpu.sync_copy(a_cnt.at[sl], tmp2)

        def w_row(k, _):
            sv = tmp[pl.ds(k * 16, 16)]
            cv = tmp2[pl.ds(k * 16, 16)]
            tmp[pl.ds(k * 16, 16)] = sv / jnp.maximum(cv, 1.0)
            return 0
        lax.fori_loop(0, SLICE // 16, w_row, 0)
        pltpu.sync_copy(tmp, a_s.at[sl])
        plsc.subcore_barrier()

    with jax.named_scope("ph5_gsc"):
        pltpu.sync_copy(a_s.at[eidx], vals)
        pltpu.sync_copy(vals, a_c.at[nidx], add=True)
        plsc.subcore_barrier()

    with jax.named_scope("ph6_cvec"):
        pltpu.sync_copy(a_c.at[pl.ds(wid * NPW, NPW)], cvec)
        rows_cp.wait()

    def grp_fma(m, accs):
        cw = cvec[pl.ds(m * 16, 16)]
        for r in range(16):
            cb = jnp.full((16,), cw[r], jnp.float32)
            i = m * 16 + r
            accs = tuple(accs[kk] + cb * rows[i, pl.ds(kk * 16, 16)]
                         for kk in range(8))
        return accs
    with jax.named_scope("ph6_fma"):
        accs = lax.fori_loop(0, NPW // 16, grp_fma,
                             tuple(zero16 for _ in range(8)))
        for kk in range(8):
            accv[pl.ds(kk * 16, 16)] = accs[kk]

        pltpu.sync_copy(accv, out_hbm.at[wid])


@functools.cache
def _sc_compute():
  return functools.partial(
    pl.kernel,
    out_type=jax.ShapeDtypeStruct((NW, EMB), jnp.float32),
    mesh=plsc.VectorSubcoreMesh(core_axis_name="c", subcore_axis_name="s",
                                num_cores=NC, num_subcores=NS),
    scratch_types=[
        pltpu.VMEM((EPT,), jnp.int32),
        pltpu.VMEM((EPT,), jnp.int32),
        pltpu.VMEM((EPT,), jnp.float32),
        pltpu.VMEM((SLICE,), jnp.float32),
        pltpu.VMEM((SLICE,), jnp.float32),
        pltpu.VMEM((NPW,), jnp.int32),
        pltpu.VMEM((NPW, EMB), jnp.float32),
        pltpu.VMEM((NPW,), jnp.float32),
        pltpu.VMEM((EMB,), jnp.float32),
        pltpu.VMEM_SHARED((NB,), jnp.float32),
        pltpu.VMEM_SHARED((NB,), jnp.float32),
        pltpu.VMEM_SHARED((NB,), jnp.float32),
        pltpu.VMEM_SHARED((NB,), jnp.float32),
        pltpu.SemaphoreType.DMA,
        pltpu.SemaphoreType.DMA,
    ],
  )(_sc_body)


ROWS_BLK = 10000
N_BLKS = N_ENTITY // ROWS_BLK


def _tc_body(pv_ref, th_ref, cb_ref, tb_ref, bb_ref, o_ref, u_ref):
    @pl.when(pl.program_id(0) == 0)
    def _():
        v = jnp.sum(pv_ref[...], axis=0, keepdims=True)
        u = jnp.dot(v, th_ref[...], preferred_element_type=jnp.float32)
        u_ref[...] = u * (1.0 / N_NODES) + cb_ref[...][None, :]
    s2 = jax.lax.dot_general(u_ref[...], tb_ref[...],
                             (((1,), (1,)), ((), ())),
                             preferred_element_type=jnp.float32)
    o_ref[...] = s2[None] + bb_ref[...]


_tc_scores = pl.pallas_call(
    _tc_body,
    grid=(N_BLKS,),
    in_specs=[
        pl.BlockSpec((NW, EMB), lambda i: (0, 0)),
        pl.BlockSpec((EMB, EMB), lambda i: (0, 0)),
        pl.BlockSpec((EMB,), lambda i: (0,)),
        pl.BlockSpec((ROWS_BLK, EMB), lambda i: (i, 0)),
        pl.BlockSpec((1, 1, ROWS_BLK), lambda i: (i, 0, 0)),
    ],
    out_specs=pl.BlockSpec((1, 1, ROWS_BLK), lambda i: (i, 0, 0)),
    out_shape=jax.ShapeDtypeStruct((N_BLKS, 1, ROWS_BLK), jnp.float32),
    scratch_shapes=[pltpu.VMEM((1, EMB), jnp.float32)],
)


def kernel(node_ids, hyper_edge_index, entity_table, theta, conv_bias, rec_bias_b):
    hei_flat = hyper_edge_index.astype(jnp.int32).reshape(2 * E_TOTAL)
    partials = _sc_compute()(hei_flat, node_ids.astype(jnp.int32),
                             entity_table)
    bb2 = rec_bias_b.reshape(N_BLKS, 1, ROWS_BLK)
    scores2 = _tc_scores(partials, theta, conv_bias, entity_table, bb2)
    return scores2.reshape(N_ENTITY)
